# baseline (speedup 1.0000x reference)
import jax
import jax.numpy as jnp
from jax import lax
from jax.experimental import pallas as pl
from jax.experimental.pallas import tpu as pltpu
from jax.experimental.pallas import tpu_sc as plsc

_VOCABS = [1000000, 1000000] + [100000] * 8 + [1000] * 16
_EMBED_DIM = 32
_BATCH = 16384
_NUM_FEATS = 26
_OUT_W = 832
_NSWEEP = 10
_NSMALL = 16

_info = plsc.get_sparse_core_info()
_NC, _NS = _info.num_cores, _info.num_subcores
_NW = _NC * _NS
_BPW = _BATCH // _NW
_IDX_ROWS = _NUM_FEATS * _BPW // 128

_AEND = [128 * (v // 128) for v in _VOCABS[:_NSWEEP]]
_TAIL = [_VOCABS[f] - _AEND[f] for f in range(_NSWEEP)]
_TBASE = [0]
for _f in range(1, _NSWEEP):
    _TBASE.append(_TBASE[-1] + _TAIL[_f - 1])
_TAIL_ROWS = _TBASE[-1] + _TAIL[-1]

_CV = 512
_CAP = 4096
_CAPC = _CAP - 32
_SCHUNK = 64
_HS = 128


def _body(*refs):
    tabs_t = refs[0:_NSWEEP]
    smalls = refs[_NSWEEP:_NSWEEP + _NSMALL]
    tail_all = refs[26]
    idx_hbm = refs[27]
    idx_scan = refs[28]
    drain16 = refs[29]
    drain32 = refs[30]
    out = refs[31]
    (idx_v, vbuf_s, sbuf, hit_r, hit_b, ch_r, ch_b, cbuf,
     hstage, drain16_v, drain32_v, gsem0, gsem1, osem, tsem) = refs[32:]

    wid = lax.axis_index("s") * _NC + lax.axis_index("c")
    base = wid * _BPW
    gsems = (gsem0, gsem1)
    iota = lax.iota(jnp.int32, 16)

    pltpu.sync_copy(idx_hbm.at[wid], idx_v)

    @pl.loop(0, _BPW // _SCHUNK)
    def _chunk(c):
        row0 = c * _SCHUNK
        for f in range(_NSWEEP, _NUM_FEATS):
            blk = 8 * f + c
            irow = blk // 2
            icol = 64 * (blk % 2)
            sem = gsems[f % 2]

            @pl.loop(0, 4)
            def _grp(g):
                iv = idx_v[irow, pl.ds(icol + g * 16, 16)]
                for j in range(16):
                    r = iv[j]
                    pltpu.async_copy(
                        smalls[f - _NSWEEP].at[r],
                        vbuf_s.at[g * 16 + j,
                                  pl.ds((f - _NSWEEP) * _EMBED_DIM,
                                        _EMBED_DIM)],
                        sem,
                    )

            if f > _NSWEEP:
                pltpu.make_async_copy(drain16, drain16_v,
                                      gsems[(f - 1) % 2]).wait()
        pltpu.make_async_copy(drain16, drain16_v,
                              gsems[(_NUM_FEATS - 1) % 2]).wait()

        @pl.loop(0, _SCHUNK)
        def _orow(b2):
            @pl.loop(0, _NSMALL)
            def _of(fo):
                pltpu.async_copy(
                    vbuf_s.at[b2, pl.ds(fo * _EMBED_DIM, _EMBED_DIM)],
                    out.at[base + row0 + b2,
                           pl.ds((_NSWEEP + fo) * _EMBED_DIM, _EMBED_DIM)],
                    osem,
                )
        for _ in range(8):
            pltpu.make_async_copy(drain32, drain32_v, osem).wait()

    @pl.loop(0, _NSWEEP)
    def _tail(fi):
        ts = jnp.where(fi < 2, _AEND[0], _AEND[2])
        tb = jnp.where(fi < 2, 64 * fi, 128 + 32 * (fi - 2))

        @pl.loop(0, 32)
        def _tl(t):
            irow = 4 * fi + t // 8
            icol = 16 * (t % 8)
            iv = idx_v[irow, pl.ds(icol, 16)]
            msk = iv >= ts
            pc = plsc.all_reduce_population_count(msk)[0]

            @pl.when(pc > 0)
            def _():
                for l in range(16):
                    r = iv[l]
                    b = base + t * 16 + l

                    @pl.when(r >= ts)
                    def _():
                        pltpu.async_copy(tail_all.at[tb + (r - ts)],
                                         hstage.at[0], tsem).wait()
                        pltpu.async_copy(
                            hstage.at[0],
                            out.at[b, pl.ds(fi * _EMBED_DIM, _EMBED_DIM)],
                            tsem).wait()

    @pl.loop(0, _NSWEEP)
    def _sweep(fi):
        tiles = jnp.where(fi < 2, _AEND[0] // 128, _AEND[2] // 128)
        aend = tiles * 128
        lo = ((wid * tiles) // _NW) * 128
        hi = (((wid + 1) * tiles) // _NW) * 128
        ncols = hi - lo
        nch = (ncols + _CV - 1) // _CV

        pltpu.sync_copy(idx_scan.at[fi], sbuf)

        def _scan_row(row, cnt):
            @pl.loop(0, 8, init_carry=cnt)
            def _scan_grp(m, cnt):
                iv = sbuf[row, pl.ds(m * 16, 16)]
                msk = (iv >= lo) & (iv < hi)
                pc = plsc.all_reduce_population_count(msk)[0]
                bvec = row * 128 + m * 16 + iota

                @pl.when(pc > 0)
                def _():
                    cc = jnp.minimum(cnt, _CAPC)
                    plsc.store_compressed(hit_r.at[pl.ds(cc, 16)], iv,
                                          mask=msk)
                    plsc.store_compressed(hit_b.at[pl.ds(cc, 16)], bvec,
                                          mask=msk)
                return cnt + pc
            return _scan_grp

        cnt = pl.loop(0, 128, init_carry=jnp.int32(0))(_scan_row)
        nhv = (cnt + 15) // 16

        @pl.loop(0, nch)
        def _chunk2(k):
            v0 = lo + k * _CV
            vhi = jnp.minimum(v0 + _CV, hi)
            v0c = jnp.minimum(v0, aend - _CV)

            for fs in range(_NSWEEP):
                @pl.when(fi == fs)
                def _():
                    pltpu.async_copy(
                        tabs_t[fs].at[:, pl.ds(v0c, _CV)], cbuf,
                        gsem0).wait()

            def _compress(hv, cc):
                rv = hit_r[pl.ds(hv * 16, 16)]
                bv = hit_b[pl.ds(hv * 16, 16)]
                m2 = ((hv * 16 + iota) < cnt) & (rv >= v0) & (rv < vhi)
                pc2 = plsc.all_reduce_population_count(m2)[0]

                @pl.when(pc2 > 0)
                def _():
                    cx = jnp.minimum(cc, _CAPC)
                    plsc.store_compressed(ch_r.at[pl.ds(cx, 16)], rv,
                                          mask=m2)
                    plsc.store_compressed(ch_b.at[pl.ds(cx, 16)], bv,
                                          mask=m2)
                return cc + pc2

            cc = pl.loop(0, nhv, init_carry=jnp.int32(0))(_compress)
            nvv = (cc + 15) // 16

            @pl.loop(0, nvv)
            def _extract(dv):
                @pl.when((dv > 0) & (dv % 8 == 0))
                def _():
                    pltpu.make_async_copy(drain32, drain32_v, osem).wait()
                rv2 = ch_r[pl.ds(dv * 16, 16)]
                bv2 = ch_b[pl.ds(dv * 16, 16)]
                for l in range(16):
                    @pl.when((dv * 16 + l) < cc)
                    def _():
                        r = rv2[l]
                        b = bv2[l]
                        csp = jnp.full((16,), 0, jnp.int32) + (r - v0c)
                        x0 = plsc.load_gather(cbuf, [iota, csp])
                        x1 = plsc.load_gather(cbuf, [iota + 16, csp])
                        h = (dv * 16 + l) % _HS
                        hstage[h, pl.ds(0, 16)] = x0
                        hstage[h, pl.ds(16, 16)] = x1
                        pltpu.async_copy(
                            hstage.at[h],
                            out.at[b, pl.ds(fi * _EMBED_DIM, _EMBED_DIM)],
                            osem)

            md = jnp.maximum(nvv - 1, 0) // 8
            rem = cc - 128 * md

            @pl.loop(0, rem)
            def _dr(i):
                pltpu.make_async_copy(out.at[0, pl.ds(0, _EMBED_DIM)],
                                      hstage.at[0], osem).wait()


@jax.jit
def _encoder(tabs_t, smalls, tail_all, idx_all, idx_scan, drain16, drain32):
    grid_kernel = pl.kernel(
        _body,
        out_type=jax.ShapeDtypeStruct((_BATCH, _OUT_W), jnp.float32),
        mesh=plsc.VectorSubcoreMesh(core_axis_name="c", subcore_axis_name="s"),
        compiler_params=pltpu.CompilerParams(needs_layout_passes=False),
        scratch_types=[
            pltpu.VMEM((_IDX_ROWS, 128), jnp.int32),
            pltpu.VMEM((_SCHUNK, _NSMALL * _EMBED_DIM), jnp.float32),
            pltpu.VMEM((128, 128), jnp.int32),
            pltpu.VMEM((_CAP,), jnp.int32),
            pltpu.VMEM((_CAP,), jnp.int32),
            pltpu.VMEM((_CAP,), jnp.int32),
            pltpu.VMEM((_CAP,), jnp.int32),
            pltpu.VMEM((32, _CV), jnp.float32),
            pltpu.VMEM((_HS, _EMBED_DIM), jnp.float32),
            pltpu.VMEM((16, 128), jnp.float32),
            pltpu.VMEM((32, 128), jnp.float32),
            pltpu.SemaphoreType.DMA,
            pltpu.SemaphoreType.DMA,
            pltpu.SemaphoreType.DMA,
            pltpu.SemaphoreType.DMA,
        ],
    )
    return grid_kernel(*tabs_t, *smalls, tail_all, idx_all, idx_scan,
                       drain16, drain32)


def kernel(table_0, table_1, table_2, table_3, table_4, table_5, table_6,
           table_7, table_8, table_9, table_10, table_11, table_12, table_13,
           table_14, table_15, table_16, table_17, table_18, table_19,
           table_20, table_21, table_22, table_23, table_24, table_25,
           idx_0, idx_1, idx_2, idx_3, idx_4, idx_5, idx_6, idx_7, idx_8,
           idx_9, idx_10, idx_11, idx_12, idx_13, idx_14, idx_15, idx_16,
           idx_17, idx_18, idx_19, idx_20, idx_21, idx_22, idx_23, idx_24,
           idx_25):
    tables = (table_0, table_1, table_2, table_3, table_4, table_5, table_6,
              table_7, table_8, table_9, table_10, table_11, table_12,
              table_13, table_14, table_15, table_16, table_17, table_18,
              table_19, table_20, table_21, table_22, table_23, table_24,
              table_25)
    idxs = (idx_0, idx_1, idx_2, idx_3, idx_4, idx_5, idx_6, idx_7, idx_8,
            idx_9, idx_10, idx_11, idx_12, idx_13, idx_14, idx_15, idx_16,
            idx_17, idx_18, idx_19, idx_20, idx_21, idx_22, idx_23, idx_24,
            idx_25)
    tabs_t = tuple(jnp.transpose(tables[f]) for f in range(_NSWEEP))
    smalls = tuple(tables[f] for f in range(_NSWEEP, _NUM_FEATS))
    tail_all = jnp.concatenate(
        [tables[f][_AEND[f]:] for f in range(_NSWEEP)], axis=0)
    idx_all = jnp.transpose(
        jnp.stack(idxs).reshape(_NUM_FEATS, _NW, _BPW), (1, 0, 2)
    ).reshape(_NW, _IDX_ROWS, 128)
    idx_scan = jnp.stack(idxs[:_NSWEEP]).reshape(_NSWEEP, 128, 128)
    drain16 = jnp.zeros((16, 128), jnp.float32)
    drain32 = jnp.zeros((32, 128), jnp.float32)
    return _encoder(tabs_t, smalls, tail_all, idx_all, idx_scan,
                    drain16, drain32)


# speedup vs baseline: 1.2521x; 1.2521x over previous
import jax
import jax.numpy as jnp
from jax import lax
from jax.experimental import pallas as pl
from jax.experimental.pallas import tpu as pltpu
from jax.experimental.pallas import tpu_sc as plsc

_VOCABS = [1000000, 1000000] + [100000] * 8 + [1000] * 16
_EMBED_DIM = 32
_BATCH = 16384
_NUM_FEATS = 26
_OUT_W = 832
_NSWEEP = 10
_NSMALL = 16

_info = plsc.get_sparse_core_info()
_NC, _NS = _info.num_cores, _info.num_subcores
_NW = _NC * _NS
_BPW = _BATCH // _NW
_IDX_ROWS = _NUM_FEATS * _BPW // 128

_AEND = [128 * (v // 128) for v in _VOCABS[:_NSWEEP]]
_TAIL = [_VOCABS[f] - _AEND[f] for f in range(_NSWEEP)]
_TBASE = [0]
for _f in range(1, _NSWEEP):
    _TBASE.append(_TBASE[-1] + _TAIL[_f - 1])
_TAIL_ROWS = _TBASE[-1] + _TAIL[-1]

_CV = 512
_CAP = 2048
_CAPC = _CAP - 32
_SCHUNK = 64
_HS = 128


def _body(*refs):
    tabs_t = refs[0:_NSWEEP]
    smalls = refs[_NSWEEP:_NSWEEP + _NSMALL]
    tail_all = refs[26]
    idx_hbm = refs[27]
    idx_scan = refs[28]
    drain16 = refs[29]
    drain32 = refs[30]
    out = refs[31]
    (idx_v, vbuf_s, sbuf, hit_r, hit_b, ch_r, ch_b, cbuf, cbuf1,
     hstage, drain16_v, drain32_v, gsem0, gsem1, osem, tsem) = refs[32:]

    wid = lax.axis_index("s") * _NC + lax.axis_index("c")
    base = wid * _BPW
    gsems = (gsem0, gsem1)
    iota = lax.iota(jnp.int32, 16)

    pltpu.sync_copy(idx_hbm.at[wid], idx_v)

    @pl.loop(0, _BPW // _SCHUNK)
    def _chunk(c):
        row0 = c * _SCHUNK
        for f in range(_NSWEEP, _NUM_FEATS):
            blk = 8 * f + c
            irow = blk // 2
            icol = 64 * (blk % 2)
            sem = gsems[f % 2]

            @pl.loop(0, 4)
            def _grp(g):
                iv = idx_v[irow, pl.ds(icol + g * 16, 16)]
                for j in range(16):
                    r = iv[j]
                    pltpu.async_copy(
                        smalls[f - _NSWEEP].at[r],
                        vbuf_s.at[g * 16 + j,
                                  pl.ds((f - _NSWEEP) * _EMBED_DIM,
                                        _EMBED_DIM)],
                        sem,
                    )

            if f > _NSWEEP:
                pltpu.make_async_copy(drain16, drain16_v,
                                      gsems[(f - 1) % 2]).wait()
        pltpu.make_async_copy(drain16, drain16_v,
                              gsems[(_NUM_FEATS - 1) % 2]).wait()

        @pl.loop(0, _SCHUNK)
        def _orow(b2):
            @pl.loop(0, _NSMALL)
            def _of(fo):
                pltpu.async_copy(
                    vbuf_s.at[b2, pl.ds(fo * _EMBED_DIM, _EMBED_DIM)],
                    out.at[base + row0 + b2,
                           pl.ds((_NSWEEP + fo) * _EMBED_DIM, _EMBED_DIM)],
                    osem,
                )
        for _ in range(8):
            pltpu.make_async_copy(drain32, drain32_v, osem).wait()

    @pl.loop(0, _NSWEEP)
    def _tail(fi):
        ts = jnp.where(fi < 2, _AEND[0], _AEND[2])
        tb = jnp.where(fi < 2, 64 * fi, 128 + 32 * (fi - 2))

        @pl.loop(0, 32)
        def _tl(t):
            irow = 4 * fi + t // 8
            icol = 16 * (t % 8)
            iv = idx_v[irow, pl.ds(icol, 16)]
            msk = iv >= ts
            pc = plsc.all_reduce_population_count(msk)[0]

            @pl.when(pc > 0)
            def _():
                for l in range(16):
                    r = iv[l]
                    b = base + t * 16 + l

                    @pl.when(r >= ts)
                    def _():
                        pltpu.async_copy(tail_all.at[tb + (r - ts)],
                                         hstage.at[0], tsem).wait()
                        pltpu.async_copy(
                            hstage.at[0],
                            out.at[b, pl.ds(fi * _EMBED_DIM, _EMBED_DIM)],
                            tsem).wait()

    @pl.loop(0, _NSWEEP)
    def _sweep(fi):
        tiles = jnp.where(fi < 2, _AEND[0] // 128, _AEND[2] // 128)
        aend = tiles * 128
        lo = ((wid * tiles) // _NW) * 128
        hi = (((wid + 1) * tiles) // _NW) * 128
        ncols = hi - lo
        nch = (ncols + _CV - 1) // _CV

        pltpu.sync_copy(idx_scan.at[fi], sbuf)

        def _scan_row(row, cnt):
            for m in range(8):
                iv = sbuf[row, pl.ds(m * 16, 16)]
                msk = (iv >= lo) & (iv < hi)
                pc = plsc.all_reduce_population_count(msk)[0]
                bvec = row * 128 + m * 16 + iota

                @pl.when(pc > 0)
                def _(cnt=cnt):
                    cc = jnp.minimum(cnt, _CAPC)
                    plsc.store_compressed(hit_r.at[pl.ds(cc, 16)], iv,
                                          mask=msk)
                    plsc.store_compressed(hit_b.at[pl.ds(cc, 16)], bvec,
                                          mask=msk)
                cnt = cnt + pc
            return cnt

        cnt = pl.loop(0, 128, init_carry=jnp.int32(0))(_scan_row)
        nhv = (cnt + 15) // 16

        def _startf(k, buf, sem):
            v0c = jnp.minimum(lo + k * _CV, aend - _CV)
            for fs in range(_NSWEEP):
                @pl.when(fi == fs)
                def _():
                    pltpu.async_copy(tabs_t[fs].at[:, pl.ds(v0c, _CV)],
                                     buf, sem)

        def _waitf(k, buf, sem):
            v0c = jnp.minimum(lo + k * _CV, aend - _CV)
            for fs in range(_NSWEEP):
                @pl.when(fi == fs)
                def _():
                    pltpu.make_async_copy(
                        tabs_t[fs].at[:, pl.ds(v0c, _CV)], buf, sem).wait()

        def _process(k, buf):
            v0 = lo + k * _CV
            vhi = jnp.minimum(v0 + _CV, hi)
            v0c = jnp.minimum(v0, aend - _CV)

            def _compress(hv, cc):
                rv = hit_r[pl.ds(hv * 16, 16)]
                bv = hit_b[pl.ds(hv * 16, 16)]
                m2 = ((hv * 16 + iota) < cnt) & (rv >= v0) & (rv < vhi)
                pc2 = plsc.all_reduce_population_count(m2)[0]

                @pl.when(pc2 > 0)
                def _():
                    cx = jnp.minimum(cc, _CAPC)
                    plsc.store_compressed(ch_r.at[pl.ds(cx, 16)], rv,
                                          mask=m2)
                    plsc.store_compressed(ch_b.at[pl.ds(cx, 16)], bv,
                                          mask=m2)
                return cc + pc2

            cc = pl.loop(0, nhv, init_carry=jnp.int32(0))(_compress)
            nvv = (cc + 15) // 16

            @pl.loop(0, nvv)
            def _extract(dv):
                @pl.when((dv > 0) & (dv % 8 == 0))
                def _():
                    pltpu.make_async_copy(drain32, drain32_v, osem).wait()
                rv2 = ch_r[pl.ds(dv * 16, 16)]
                bv2 = ch_b[pl.ds(dv * 16, 16)]
                for l in range(16):
                    @pl.when((dv * 16 + l) < cc)
                    def _():
                        r = rv2[l]
                        b = bv2[l]
                        csp = jnp.full((16,), 0, jnp.int32) + (r - v0c)
                        x0 = plsc.load_gather(buf, [iota, csp])
                        x1 = plsc.load_gather(buf, [iota + 16, csp])
                        h = (dv * 16 + l) % _HS
                        hstage[h, pl.ds(0, 16)] = x0
                        hstage[h, pl.ds(16, 16)] = x1
                        pltpu.async_copy(
                            hstage.at[h],
                            out.at[b, pl.ds(fi * _EMBED_DIM, _EMBED_DIM)],
                            osem)

            md = jnp.maximum(nvv - 1, 0) // 8
            rem = cc - 128 * md

            @pl.loop(0, rem)
            def _dr(i):
                pltpu.make_async_copy(out.at[0, pl.ds(0, _EMBED_DIM)],
                                      hstage.at[0], osem).wait()

        _startf(0, cbuf, gsem0)

        @pl.loop(0, (nch + 1) // 2)
        def _pair(kk):
            k0 = 2 * kk

            @pl.when(k0 + 1 < nch)
            def _():
                _startf(k0 + 1, cbuf1, gsem1)
            _waitf(k0, cbuf, gsem0)
            _process(k0, cbuf)

            @pl.when(k0 + 2 < nch)
            def _():
                _startf(k0 + 2, cbuf, gsem0)

            @pl.when(k0 + 1 < nch)
            def _():
                _waitf(k0 + 1, cbuf1, gsem1)
                _process(k0 + 1, cbuf1)


@jax.jit
def _encoder(tabs_t, smalls, tail_all, idx_all, idx_scan, drain16, drain32):
    grid_kernel = pl.kernel(
        _body,
        out_type=jax.ShapeDtypeStruct((_BATCH, _OUT_W), jnp.float32),
        mesh=plsc.VectorSubcoreMesh(core_axis_name="c", subcore_axis_name="s"),
        compiler_params=pltpu.CompilerParams(needs_layout_passes=False),
        scratch_types=[
            pltpu.VMEM((_IDX_ROWS, 128), jnp.int32),
            pltpu.VMEM((_SCHUNK, _NSMALL * _EMBED_DIM), jnp.float32),
            pltpu.VMEM((128, 128), jnp.int32),
            pltpu.VMEM((_CAP,), jnp.int32),
            pltpu.VMEM((_CAP,), jnp.int32),
            pltpu.VMEM((_CAP,), jnp.int32),
            pltpu.VMEM((_CAP,), jnp.int32),
            pltpu.VMEM((32, _CV), jnp.float32),
            pltpu.VMEM((32, _CV), jnp.float32),
            pltpu.VMEM((_HS, _EMBED_DIM), jnp.float32),
            pltpu.VMEM((16, 128), jnp.float32),
            pltpu.VMEM((32, 128), jnp.float32),
            pltpu.SemaphoreType.DMA,
            pltpu.SemaphoreType.DMA,
            pltpu.SemaphoreType.DMA,
            pltpu.SemaphoreType.DMA,
        ],
    )
    return grid_kernel(*tabs_t, *smalls, tail_all, idx_all, idx_scan,
                       drain16, drain32)


def kernel(table_0, table_1, table_2, table_3, table_4, table_5, table_6,
           table_7, table_8, table_9, table_10, table_11, table_12, table_13,
           table_14, table_15, table_16, table_17, table_18, table_19,
           table_20, table_21, table_22, table_23, table_24, table_25,
           idx_0, idx_1, idx_2, idx_3, idx_4, idx_5, idx_6, idx_7, idx_8,
           idx_9, idx_10, idx_11, idx_12, idx_13, idx_14, idx_15, idx_16,
           idx_17, idx_18, idx_19, idx_20, idx_21, idx_22, idx_23, idx_24,
           idx_25):
    tables = (table_0, table_1, table_2, table_3, table_4, table_5, table_6,
              table_7, table_8, table_9, table_10, table_11, table_12,
              table_13, table_14, table_15, table_16, table_17, table_18,
              table_19, table_20, table_21, table_22, table_23, table_24,
              table_25)
    idxs = (idx_0, idx_1, idx_2, idx_3, idx_4, idx_5, idx_6, idx_7, idx_8,
            idx_9, idx_10, idx_11, idx_12, idx_13, idx_14, idx_15, idx_16,
            idx_17, idx_18, idx_19, idx_20, idx_21, idx_22, idx_23, idx_24,
            idx_25)
    tabs_t = tuple(jnp.transpose(tables[f]) for f in range(_NSWEEP))
    smalls = tuple(tables[f] for f in range(_NSWEEP, _NUM_FEATS))
    tail_all = jnp.concatenate(
        [tables[f][_AEND[f]:] for f in range(_NSWEEP)], axis=0)
    idx_all = jnp.transpose(
        jnp.stack(idxs).reshape(_NUM_FEATS, _NW, _BPW), (1, 0, 2)
    ).reshape(_NW, _IDX_ROWS, 128)
    idx_scan = jnp.stack(idxs[:_NSWEEP]).reshape(_NSWEEP, 128, 128)
    drain16 = jnp.zeros((16, 128), jnp.float32)
    drain32 = jnp.zeros((32, 128), jnp.float32)
    return _encoder(tabs_t, smalls, tail_all, idx_all, idx_scan,
                    drain16, drain32)


# speedup vs baseline: 1.2921x; 1.0319x over previous
import jax
import jax.numpy as jnp
from jax import lax
from jax.experimental import pallas as pl
from jax.experimental.pallas import tpu as pltpu
from jax.experimental.pallas import tpu_sc as plsc

_VOCABS = [1000000, 1000000] + [100000] * 8 + [1000] * 16
_EMBED_DIM = 32
_BATCH = 16384
_NUM_FEATS = 26
_OUT_W = 832
_NSWEEP = 10
_NSMALL = 16

_info = plsc.get_sparse_core_info()
_NC, _NS = _info.num_cores, _info.num_subcores
_NW = _NC * _NS
_BPW = _BATCH // _NW
_IDX_ROWS = _NUM_FEATS * _BPW // 128

_AEND = [128 * (v // 128) for v in _VOCABS[:_NSWEEP]]
_TAIL = [_VOCABS[f] - _AEND[f] for f in range(_NSWEEP)]
_TBASE = [0]
for _f in range(1, _NSWEEP):
    _TBASE.append(_TBASE[-1] + _TAIL[_f - 1])
_TAIL_ROWS = _TBASE[-1] + _TAIL[-1]

_CV = 512
_CAP = 2048
_CAPC = _CAP - 32
_SCHUNK = 64
_HS = 128


def _body(*refs):
    tabs_t = refs[0:_NSWEEP]
    smalls = refs[_NSWEEP:_NSWEEP + _NSMALL]
    tail_all = refs[26]
    idx_hbm = refs[27]
    idx_scan = refs[28]
    drain16 = refs[29]
    drain32 = refs[30]
    out = refs[31]
    (idx_v, vbuf_s, sbuf, hit_r, hit_b, ch_r, ch_b, cbuf, cbuf1,
     hstage, drain16_v, drain32_v, gsem0, gsem1, osem, tsem) = refs[32:]

    wid = lax.axis_index("s") * _NC + lax.axis_index("c")
    base = wid * _BPW
    gsems = (gsem0, gsem1)
    iota = lax.iota(jnp.int32, 16)

    pltpu.sync_copy(idx_hbm.at[wid], idx_v)

    @pl.loop(0, _BPW // _SCHUNK)
    def _chunk(c):
        row0 = c * _SCHUNK
        for f in range(_NSWEEP, _NUM_FEATS):
            blk = 8 * f + c
            irow = blk // 2
            icol = 64 * (blk % 2)
            sem = gsems[f % 2]

            @pl.loop(0, 4)
            def _grp(g):
                iv = idx_v[irow, pl.ds(icol + g * 16, 16)]
                for j in range(16):
                    r = iv[j]
                    pltpu.async_copy(
                        smalls[f - _NSWEEP].at[r],
                        vbuf_s.at[g * 16 + j,
                                  pl.ds((f - _NSWEEP) * _EMBED_DIM,
                                        _EMBED_DIM)],
                        sem,
                    )

            if f > _NSWEEP:
                pltpu.make_async_copy(drain16, drain16_v,
                                      gsems[(f - 1) % 2]).wait()
        pltpu.make_async_copy(drain16, drain16_v,
                              gsems[(_NUM_FEATS - 1) % 2]).wait()

        @pl.loop(0, _SCHUNK)
        def _orow(b2):
            @pl.loop(0, 8)
            def _of(fo):
                pltpu.async_copy(
                    vbuf_s.at[b2, pl.ds(fo * 64, 64)],
                    out.at[base + row0 + b2,
                           pl.ds(_NSWEEP * _EMBED_DIM + fo * 64, 64)],
                    osem,
                )
        for _ in range(8):
            pltpu.make_async_copy(drain32, drain32_v, osem).wait()

    @pl.loop(0, _NSWEEP)
    def _tail(fi):
        ts = jnp.where(fi < 2, _AEND[0], _AEND[2])
        tb = jnp.where(fi < 2, 64 * fi, 128 + 32 * (fi - 2))

        @pl.loop(0, 32)
        def _tl(t):
            irow = 4 * fi + t // 8
            icol = 16 * (t % 8)
            iv = idx_v[irow, pl.ds(icol, 16)]
            msk = iv >= ts
            pc = plsc.all_reduce_population_count(msk)[0]

            @pl.when(pc > 0)
            def _():
                for l in range(16):
                    r = iv[l]
                    b = base + t * 16 + l

                    @pl.when(r >= ts)
                    def _():
                        pltpu.async_copy(tail_all.at[tb + (r - ts)],
                                         hstage.at[0], tsem).wait()
                        pltpu.async_copy(
                            hstage.at[0],
                            out.at[b, pl.ds(fi * _EMBED_DIM, _EMBED_DIM)],
                            tsem).wait()

    @pl.loop(0, _NSWEEP)
    def _sweep(fi):
        tiles = jnp.where(fi < 2, _AEND[0] // 128, _AEND[2] // 128)
        aend = tiles * 128
        lo = ((wid * tiles) // _NW) * 128
        hi = (((wid + 1) * tiles) // _NW) * 128
        ncols = hi - lo
        nch = (ncols + _CV - 1) // _CV

        pltpu.sync_copy(idx_scan.at[fi], sbuf)

        def _scan_row(row, cnt):
            for m in range(8):
                iv = sbuf[row, pl.ds(m * 16, 16)]
                msk = (iv >= lo) & (iv < hi)
                pc = plsc.all_reduce_population_count(msk)[0]
                bvec = row * 128 + m * 16 + iota

                @pl.when(pc > 0)
                def _(cnt=cnt):
                    cc = jnp.minimum(cnt, _CAPC)
                    plsc.store_compressed(hit_r.at[pl.ds(cc, 16)], iv,
                                          mask=msk)
                    plsc.store_compressed(hit_b.at[pl.ds(cc, 16)], bvec,
                                          mask=msk)
                cnt = cnt + pc
            return cnt

        cnt = pl.loop(0, 128, init_carry=jnp.int32(0))(_scan_row)
        nhv = (cnt + 15) // 16

        def _startf(k, buf, sem):
            v0c = jnp.minimum(lo + k * _CV, aend - _CV)
            for fs in range(_NSWEEP):
                @pl.when(fi == fs)
                def _():
                    pltpu.async_copy(tabs_t[fs].at[:, pl.ds(v0c, _CV)],
                                     buf, sem)

        def _waitf(k, buf, sem):
            v0c = jnp.minimum(lo + k * _CV, aend - _CV)
            for fs in range(_NSWEEP):
                @pl.when(fi == fs)
                def _():
                    pltpu.make_async_copy(
                        tabs_t[fs].at[:, pl.ds(v0c, _CV)], buf, sem).wait()

        def _process(k, buf):
            v0 = lo + k * _CV
            vhi = jnp.minimum(v0 + _CV, hi)
            v0c = jnp.minimum(v0, aend - _CV)

            def _compress(hv, cc):
                rv = hit_r[pl.ds(hv * 16, 16)]
                bv = hit_b[pl.ds(hv * 16, 16)]
                m2 = ((hv * 16 + iota) < cnt) & (rv >= v0) & (rv < vhi)
                pc2 = plsc.all_reduce_population_count(m2)[0]

                @pl.when(pc2 > 0)
                def _():
                    cx = jnp.minimum(cc, _CAPC)
                    plsc.store_compressed(ch_r.at[pl.ds(cx, 16)], rv,
                                          mask=m2)
                    plsc.store_compressed(ch_b.at[pl.ds(cx, 16)], bv,
                                          mask=m2)
                return cc + pc2

            cc = pl.loop(0, nhv, init_carry=jnp.int32(0))(_compress)
            nvv = (cc + 15) // 16

            @pl.loop(0, nvv)
            def _extract(dv):
                @pl.when((dv > 0) & (dv % 8 == 0))
                def _():
                    pltpu.make_async_copy(drain32, drain32_v, osem).wait()
                rv2 = ch_r[pl.ds(dv * 16, 16)]
                bv2 = ch_b[pl.ds(dv * 16, 16)]
                for l in range(16):
                    @pl.when((dv * 16 + l) < cc)
                    def _():
                        r = rv2[l]
                        b = bv2[l]
                        csp = jnp.full((16,), 0, jnp.int32) + (r - v0c)
                        x0 = plsc.load_gather(buf, [iota, csp])
                        x1 = plsc.load_gather(buf, [iota + 16, csp])
                        h = (dv * 16 + l) % _HS
                        hstage[h, pl.ds(0, 16)] = x0
                        hstage[h, pl.ds(16, 16)] = x1
                        pltpu.async_copy(
                            hstage.at[h],
                            out.at[b, pl.ds(fi * _EMBED_DIM, _EMBED_DIM)],
                            osem)

            md = jnp.maximum(nvv - 1, 0) // 8
            rem = cc - 128 * md

            @pl.loop(0, rem)
            def _dr(i):
                pltpu.make_async_copy(out.at[0, pl.ds(0, _EMBED_DIM)],
                                      hstage.at[0], osem).wait()

        _startf(0, cbuf, gsem0)

        @pl.loop(0, (nch + 1) // 2)
        def _pair(kk):
            k0 = 2 * kk

            @pl.when(k0 + 1 < nch)
            def _():
                _startf(k0 + 1, cbuf1, gsem1)
            _waitf(k0, cbuf, gsem0)
            _process(k0, cbuf)

            @pl.when(k0 + 2 < nch)
            def _():
                _startf(k0 + 2, cbuf, gsem0)

            @pl.when(k0 + 1 < nch)
            def _():
                _waitf(k0 + 1, cbuf1, gsem1)
                _process(k0 + 1, cbuf1)


@jax.jit
def _encoder(tabs_t, smalls, tail_all, idx_all, idx_scan, drain16, drain32):
    grid_kernel = pl.kernel(
        _body,
        out_type=jax.ShapeDtypeStruct((_BATCH, _OUT_W), jnp.float32),
        mesh=plsc.VectorSubcoreMesh(core_axis_name="c", subcore_axis_name="s"),
        compiler_params=pltpu.CompilerParams(needs_layout_passes=False),
        scratch_types=[
            pltpu.VMEM((_IDX_ROWS, 128), jnp.int32),
            pltpu.VMEM((_SCHUNK, _NSMALL * _EMBED_DIM), jnp.float32),
            pltpu.VMEM((128, 128), jnp.int32),
            pltpu.VMEM((_CAP,), jnp.int32),
            pltpu.VMEM((_CAP,), jnp.int32),
            pltpu.VMEM((_CAP,), jnp.int32),
            pltpu.VMEM((_CAP,), jnp.int32),
            pltpu.VMEM((32, _CV), jnp.float32),
            pltpu.VMEM((32, _CV), jnp.float32),
            pltpu.VMEM((_HS, _EMBED_DIM), jnp.float32),
            pltpu.VMEM((16, 128), jnp.float32),
            pltpu.VMEM((32, 128), jnp.float32),
            pltpu.SemaphoreType.DMA,
            pltpu.SemaphoreType.DMA,
            pltpu.SemaphoreType.DMA,
            pltpu.SemaphoreType.DMA,
        ],
    )
    return grid_kernel(*tabs_t, *smalls, tail_all, idx_all, idx_scan,
                       drain16, drain32)


def kernel(table_0, table_1, table_2, table_3, table_4, table_5, table_6,
           table_7, table_8, table_9, table_10, table_11, table_12, table_13,
           table_14, table_15, table_16, table_17, table_18, table_19,
           table_20, table_21, table_22, table_23, table_24, table_25,
           idx_0, idx_1, idx_2, idx_3, idx_4, idx_5, idx_6, idx_7, idx_8,
           idx_9, idx_10, idx_11, idx_12, idx_13, idx_14, idx_15, idx_16,
           idx_17, idx_18, idx_19, idx_20, idx_21, idx_22, idx_23, idx_24,
           idx_25):
    tables = (table_0, table_1, table_2, table_3, table_4, table_5, table_6,
              table_7, table_8, table_9, table_10, table_11, table_12,
              table_13, table_14, table_15, table_16, table_17, table_18,
              table_19, table_20, table_21, table_22, table_23, table_24,
              table_25)
    idxs = (idx_0, idx_1, idx_2, idx_3, idx_4, idx_5, idx_6, idx_7, idx_8,
            idx_9, idx_10, idx_11, idx_12, idx_13, idx_14, idx_15, idx_16,
            idx_17, idx_18, idx_19, idx_20, idx_21, idx_22, idx_23, idx_24,
            idx_25)
    tabs_t = tuple(jnp.transpose(tables[f]) for f in range(_NSWEEP))
    smalls = tuple(tables[f] for f in range(_NSWEEP, _NUM_FEATS))
    tail_all = jnp.concatenate(
        [tables[f][_AEND[f]:] for f in range(_NSWEEP)], axis=0)
    idx_all = jnp.transpose(
        jnp.stack(idxs).reshape(_NUM_FEATS, _NW, _BPW), (1, 0, 2)
    ).reshape(_NW, _IDX_ROWS, 128)
    idx_scan = jnp.stack(idxs[:_NSWEEP]).reshape(_NSWEEP, 128, 128)
    drain16 = jnp.zeros((16, 128), jnp.float32)
    drain32 = jnp.zeros((32, 128), jnp.float32)
    return _encoder(tabs_t, smalls, tail_all, idx_all, idx_scan,
                    drain16, drain32)


# speedup vs baseline: 1.3479x; 1.0432x over previous
import jax
import jax.numpy as jnp
from jax import lax
from jax.experimental import pallas as pl
from jax.experimental.pallas import tpu as pltpu
from jax.experimental.pallas import tpu_sc as plsc

_VOCABS = [1000000, 1000000] + [100000] * 8 + [1000] * 16
_EMBED_DIM = 32
_BATCH = 16384
_NUM_FEATS = 26
_OUT_W = 832
_NSWEEP = 10
_NSMALL = 16

_info = plsc.get_sparse_core_info()
_NC, _NS = _info.num_cores, _info.num_subcores
_NW = _NC * _NS
_BPW = _BATCH // _NW
_IDX_ROWS = _NUM_FEATS * _BPW // 128

_AEND = [128 * (v // 128) for v in _VOCABS[:_NSWEEP]]
_TAIL = [_VOCABS[f] - _AEND[f] for f in range(_NSWEEP)]
_TBASE = [0]
for _f in range(1, _NSWEEP):
    _TBASE.append(_TBASE[-1] + _TAIL[_f - 1])
_TAIL_ROWS = _TBASE[-1] + _TAIL[-1]

_CV = 1024
_CAP = 1024
_CAPC = _CAP - 32
_SCHUNK = 32
_HS = 64


def _body(*refs):
    tabs_t = refs[0:_NSWEEP]
    smalls = refs[_NSWEEP:_NSWEEP + _NSMALL]
    tail_all = refs[26]
    idx_hbm = refs[27]
    idx_scan = refs[28]
    drain8 = refs[29]
    drain32 = refs[30]
    out = refs[31]
    (idx_v, vbuf_s, sbuf, hit_r, hit_b, ch_r, ch_b, cbuf, cbuf1,
     hstage, drain8_v, drain32_v, gsem0, gsem1, osem, tsem) = refs[32:]

    wid = lax.axis_index("s") * _NC + lax.axis_index("c")
    base = wid * _BPW
    gsems = (gsem0, gsem1)
    iota = lax.iota(jnp.int32, 16)

    pltpu.sync_copy(idx_hbm.at[wid], idx_v)

    @pl.loop(0, _BPW // _SCHUNK)
    def _chunk(c):
        row0 = c * _SCHUNK
        for f in range(_NSWEEP, _NUM_FEATS):
            blk = 16 * f + c
            irow = blk // 4
            icol = 32 * (blk % 4)
            sem = gsems[f % 2]

            @pl.loop(0, 2)
            def _grp(g):
                iv = idx_v[irow, pl.ds(icol + g * 16, 16)]
                for j in range(16):
                    r = iv[j]
                    pltpu.async_copy(
                        smalls[f - _NSWEEP].at[r],
                        vbuf_s.at[g * 16 + j,
                                  pl.ds((f - _NSWEEP) * _EMBED_DIM,
                                        _EMBED_DIM)],
                        sem,
                    )

            if f > _NSWEEP:
                pltpu.make_async_copy(drain8, drain8_v,
                                      gsems[(f - 1) % 2]).wait()
        pltpu.make_async_copy(drain8, drain8_v,
                              gsems[(_NUM_FEATS - 1) % 2]).wait()

        @pl.loop(0, _SCHUNK)
        def _orow(b2):
            @pl.loop(0, 8)
            def _of(fo):
                pltpu.async_copy(
                    vbuf_s.at[b2, pl.ds(fo * 64, 64)],
                    out.at[base + row0 + b2,
                           pl.ds(_NSWEEP * _EMBED_DIM + fo * 64, 64)],
                    osem,
                )
        for _ in range(4):
            pltpu.make_async_copy(drain32, drain32_v, osem).wait()

    @pl.loop(0, _NSWEEP)
    def _tail(fi):
        ts = jnp.where(fi < 2, _AEND[0], _AEND[2])
        tb = jnp.where(fi < 2, 64 * fi, 128 + 32 * (fi - 2))

        @pl.loop(0, 32)
        def _tl(t):
            irow = 4 * fi + t // 8
            icol = 16 * (t % 8)
            iv = idx_v[irow, pl.ds(icol, 16)]
            msk = iv >= ts
            pc = plsc.all_reduce_population_count(msk)[0]

            @pl.when(pc > 0)
            def _():
                for l in range(16):
                    r = iv[l]
                    b = base + t * 16 + l

                    @pl.when(r >= ts)
                    def _():
                        pltpu.async_copy(tail_all.at[tb + (r - ts)],
                                         hstage.at[0], tsem).wait()
                        pltpu.async_copy(
                            hstage.at[0],
                            out.at[b, pl.ds(fi * _EMBED_DIM, _EMBED_DIM)],
                            tsem).wait()

    @pl.loop(0, _NSWEEP)
    def _sweep(fi):
        tiles = jnp.where(fi < 2, _AEND[0] // 128, _AEND[2] // 128)
        aend = tiles * 128
        lo = ((wid * tiles) // _NW) * 128
        hi = (((wid + 1) * tiles) // _NW) * 128
        ncols = hi - lo
        nch = (ncols + _CV - 1) // _CV

        pltpu.sync_copy(idx_scan.at[fi], sbuf)

        def _scan_row(row, cnt):
            for m in range(8):
                iv = sbuf[row, pl.ds(m * 16, 16)]
                msk = (iv >= lo) & (iv < hi)
                pc = plsc.all_reduce_population_count(msk)[0]
                bvec = row * 128 + m * 16 + iota

                @pl.when(pc > 0)
                def _(cnt=cnt):
                    cc = jnp.minimum(cnt, _CAPC)
                    plsc.store_compressed(hit_r.at[pl.ds(cc, 16)], iv,
                                          mask=msk)
                    plsc.store_compressed(hit_b.at[pl.ds(cc, 16)], bvec,
                                          mask=msk)
                cnt = cnt + pc
            return cnt

        cnt = pl.loop(0, 128, init_carry=jnp.int32(0))(_scan_row)
        nhv = (cnt + 15) // 16

        def _startf(k, buf, sem):
            v0c = jnp.minimum(lo + k * _CV, aend - _CV)
            for fs in range(_NSWEEP):
                @pl.when(fi == fs)
                def _():
                    pltpu.async_copy(tabs_t[fs].at[:, pl.ds(v0c, _CV)],
                                     buf, sem)

        def _waitf(k, buf, sem):
            v0c = jnp.minimum(lo + k * _CV, aend - _CV)
            for fs in range(_NSWEEP):
                @pl.when(fi == fs)
                def _():
                    pltpu.make_async_copy(
                        tabs_t[fs].at[:, pl.ds(v0c, _CV)], buf, sem).wait()

        def _process(k, buf):
            v0 = lo + k * _CV
            vhi = jnp.minimum(v0 + _CV, hi)
            v0c = jnp.minimum(v0, aend - _CV)

            def _compress(hv, cc):
                rv = hit_r[pl.ds(hv * 16, 16)]
                bv = hit_b[pl.ds(hv * 16, 16)]
                m2 = ((hv * 16 + iota) < cnt) & (rv >= v0) & (rv < vhi)
                pc2 = plsc.all_reduce_population_count(m2)[0]

                @pl.when(pc2 > 0)
                def _():
                    cx = jnp.minimum(cc, _CAPC)
                    plsc.store_compressed(ch_r.at[pl.ds(cx, 16)], rv,
                                          mask=m2)
                    plsc.store_compressed(ch_b.at[pl.ds(cx, 16)], bv,
                                          mask=m2)
                return cc + pc2

            cc = pl.loop(0, nhv, init_carry=jnp.int32(0))(_compress)
            nvv = (cc + 15) // 16

            @pl.loop(0, nvv)
            def _extract(dv):
                @pl.when((dv > 0) & (dv % 4 == 0))
                def _():
                    pltpu.make_async_copy(drain8, drain8_v, osem).wait()
                    pltpu.make_async_copy(drain8, drain8_v, osem).wait()
                rv2 = ch_r[pl.ds(dv * 16, 16)]
                bv2 = ch_b[pl.ds(dv * 16, 16)]
                for l in range(16):
                    @pl.when((dv * 16 + l) < cc)
                    def _():
                        r = rv2[l]
                        b = bv2[l]
                        csp = jnp.full((16,), 0, jnp.int32) + (r - v0c)
                        x0 = plsc.load_gather(buf, [iota, csp])
                        x1 = plsc.load_gather(buf, [iota + 16, csp])
                        h = (dv * 16 + l) % _HS
                        hstage[h, pl.ds(0, 16)] = x0
                        hstage[h, pl.ds(16, 16)] = x1
                        pltpu.async_copy(
                            hstage.at[h],
                            out.at[b, pl.ds(fi * _EMBED_DIM, _EMBED_DIM)],
                            osem)

            md = jnp.maximum(nvv - 1, 0) // 4
            rem = cc - 64 * md

            @pl.loop(0, rem)
            def _dr(i):
                pltpu.make_async_copy(out.at[0, pl.ds(0, _EMBED_DIM)],
                                      hstage.at[0], osem).wait()

        _startf(0, cbuf, gsem0)

        @pl.loop(0, (nch + 1) // 2)
        def _pair(kk):
            k0 = 2 * kk

            @pl.when(k0 + 1 < nch)
            def _():
                _startf(k0 + 1, cbuf1, gsem1)
            _waitf(k0, cbuf, gsem0)
            _process(k0, cbuf)

            @pl.when(k0 + 2 < nch)
            def _():
                _startf(k0 + 2, cbuf, gsem0)

            @pl.when(k0 + 1 < nch)
            def _():
                _waitf(k0 + 1, cbuf1, gsem1)
                _process(k0 + 1, cbuf1)


@jax.jit
def _encoder(tabs_t, smalls, tail_all, idx_all, idx_scan, drain8, drain32):
    grid_kernel = pl.kernel(
        _body,
        out_type=jax.ShapeDtypeStruct((_BATCH, _OUT_W), jnp.float32),
        mesh=plsc.VectorSubcoreMesh(core_axis_name="c", subcore_axis_name="s"),
        compiler_params=pltpu.CompilerParams(needs_layout_passes=False),
        scratch_types=[
            pltpu.VMEM((_IDX_ROWS, 128), jnp.int32),
            pltpu.VMEM((_SCHUNK, _NSMALL * _EMBED_DIM), jnp.float32),
            pltpu.VMEM((128, 128), jnp.int32),
            pltpu.VMEM((_CAP,), jnp.int32),
            pltpu.VMEM((_CAP,), jnp.int32),
            pltpu.VMEM((_CAP,), jnp.int32),
            pltpu.VMEM((_CAP,), jnp.int32),
            pltpu.VMEM((32, _CV), jnp.float32),
            pltpu.VMEM((32, _CV), jnp.float32),
            pltpu.VMEM((_HS, _EMBED_DIM), jnp.float32),
            pltpu.VMEM((8, 128), jnp.float32),
            pltpu.VMEM((32, 128), jnp.float32),
            pltpu.SemaphoreType.DMA,
            pltpu.SemaphoreType.DMA,
            pltpu.SemaphoreType.DMA,
            pltpu.SemaphoreType.DMA,
        ],
    )
    return grid_kernel(*tabs_t, *smalls, tail_all, idx_all, idx_scan,
                       drain8, drain32)


def kernel(table_0, table_1, table_2, table_3, table_4, table_5, table_6,
           table_7, table_8, table_9, table_10, table_11, table_12, table_13,
           table_14, table_15, table_16, table_17, table_18, table_19,
           table_20, table_21, table_22, table_23, table_24, table_25,
           idx_0, idx_1, idx_2, idx_3, idx_4, idx_5, idx_6, idx_7, idx_8,
           idx_9, idx_10, idx_11, idx_12, idx_13, idx_14, idx_15, idx_16,
           idx_17, idx_18, idx_19, idx_20, idx_21, idx_22, idx_23, idx_24,
           idx_25):
    tables = (table_0, table_1, table_2, table_3, table_4, table_5, table_6,
              table_7, table_8, table_9, table_10, table_11, table_12,
              table_13, table_14, table_15, table_16, table_17, table_18,
              table_19, table_20, table_21, table_22, table_23, table_24,
              table_25)
    idxs = (idx_0, idx_1, idx_2, idx_3, idx_4, idx_5, idx_6, idx_7, idx_8,
            idx_9, idx_10, idx_11, idx_12, idx_13, idx_14, idx_15, idx_16,
            idx_17, idx_18, idx_19, idx_20, idx_21, idx_22, idx_23, idx_24,
            idx_25)
    tabs_t = tuple(jnp.transpose(tables[f]) for f in range(_NSWEEP))
    smalls = tuple(tables[f] for f in range(_NSWEEP, _NUM_FEATS))
    tail_all = jnp.concatenate(
        [tables[f][_AEND[f]:] for f in range(_NSWEEP)], axis=0)
    idx_all = jnp.transpose(
        jnp.stack(idxs).reshape(_NUM_FEATS, _NW, _BPW), (1, 0, 2)
    ).reshape(_NW, _IDX_ROWS, 128)
    idx_scan = jnp.stack(idxs[:_NSWEEP]).reshape(_NSWEEP, 128, 128)
    drain8 = jnp.zeros((8, 128), jnp.float32)
    drain32 = jnp.zeros((32, 128), jnp.float32)
    return _encoder(tabs_t, smalls, tail_all, idx_all, idx_scan,
                    drain8, drain32)


# speedup vs baseline: 1.3823x; 1.0255x over previous
import jax
import jax.numpy as jnp
from jax import lax
from jax.experimental import pallas as pl
from jax.experimental.pallas import tpu as pltpu
from jax.experimental.pallas import tpu_sc as plsc

_VOCABS = [1000000, 1000000] + [100000] * 8 + [1000] * 16
_EMBED_DIM = 32
_BATCH = 16384
_NUM_FEATS = 26
_OUT_W = 832
_NSWEEP = 10
_NSMALL = 16

_info = plsc.get_sparse_core_info()
_NC, _NS = _info.num_cores, _info.num_subcores
_NW = _NC * _NS
_BPW = _BATCH // _NW
_IDX_ROWS = _NUM_FEATS * _BPW // 128

_AEND = [128 * (v // 128) for v in _VOCABS[:_NSWEEP]]
_TAIL = [_VOCABS[f] - _AEND[f] for f in range(_NSWEEP)]
_TBASE = [0]
for _f in range(1, _NSWEEP):
    _TBASE.append(_TBASE[-1] + _TAIL[_f - 1])
_TAIL_ROWS = _TBASE[-1] + _TAIL[-1]

_CV = 1024
_CAP = 1024
_CAPC = _CAP - 32
_SCHUNK = 32
_HS = 64


def _body(*refs):
    tabs_t = refs[0:_NSWEEP]
    smalls = refs[_NSWEEP:_NSWEEP + _NSMALL]
    tail_all = refs[26]
    idx_hbm = refs[27]
    idx_scan = refs[28]
    drain8 = refs[29]
    drain32 = refs[30]
    out = refs[31]
    (idx_v, vbuf_s, sbuf, hit_r, hit_b, ch_r, ch_b, cbuf, cbuf1,
     hstage, drain8_v, drain32_v, gsem0, gsem1, osem, tsem) = refs[32:]

    wid = lax.axis_index("s") * _NC + lax.axis_index("c")
    base = wid * _BPW
    gsems = (gsem0, gsem1)
    iota = lax.iota(jnp.int32, 16)

    pltpu.sync_copy(idx_hbm.at[wid], idx_v)

    @pl.loop(0, _BPW // _SCHUNK)
    def _chunk(c):
        row0 = c * _SCHUNK
        for f in range(_NSWEEP, _NUM_FEATS):
            blk = 16 * f + c
            irow = blk // 4
            icol = 32 * (blk % 4)
            sem = gsems[f % 2]

            @pl.loop(0, 2)
            def _grp(g):
                iv = idx_v[irow, pl.ds(icol + g * 16, 16)]
                for j in range(16):
                    r = iv[j]
                    pltpu.async_copy(
                        smalls[f - _NSWEEP].at[r],
                        vbuf_s.at[g * 16 + j,
                                  pl.ds((f - _NSWEEP) * _EMBED_DIM,
                                        _EMBED_DIM)],
                        sem,
                    )

            if f > _NSWEEP:
                pltpu.make_async_copy(drain8, drain8_v,
                                      gsems[(f - 1) % 2]).wait()
        pltpu.make_async_copy(drain8, drain8_v,
                              gsems[(_NUM_FEATS - 1) % 2]).wait()

        @pl.loop(0, _SCHUNK)
        def _orow(b2):
            @pl.loop(0, 8)
            def _of(fo):
                pltpu.async_copy(
                    vbuf_s.at[b2, pl.ds(fo * 64, 64)],
                    out.at[base + row0 + b2,
                           pl.ds(_NSWEEP * _EMBED_DIM + fo * 64, 64)],
                    osem,
                )
        for _ in range(4):
            pltpu.make_async_copy(drain32, drain32_v, osem).wait()

    @pl.loop(0, _NSWEEP)
    def _tail(fi):
        ts = jnp.where(fi < 2, _AEND[0], _AEND[2])
        tb = jnp.where(fi < 2, 64 * fi, 128 + 32 * (fi - 2))

        @pl.loop(0, 32)
        def _tl(t):
            irow = 4 * fi + t // 8
            icol = 16 * (t % 8)
            iv = idx_v[irow, pl.ds(icol, 16)]
            msk = iv >= ts
            pc = plsc.all_reduce_population_count(msk)[0]

            @pl.when(pc > 0)
            def _():
                for l in range(16):
                    r = iv[l]
                    b = base + t * 16 + l

                    @pl.when(r >= ts)
                    def _():
                        pltpu.async_copy(tail_all.at[tb + (r - ts)],
                                         hstage.at[0], tsem).wait()
                        pltpu.async_copy(
                            hstage.at[0],
                            out.at[b, pl.ds(fi * _EMBED_DIM, _EMBED_DIM)],
                            tsem).wait()

    @pl.loop(0, _NSWEEP)
    def _sweep(fi):
        tiles = jnp.where(fi < 2, _AEND[0] // 128, _AEND[2] // 128)
        aend = tiles * 128
        lo = ((wid * tiles) // _NW) * 128
        hi = (((wid + 1) * tiles) // _NW) * 128
        ncols = hi - lo
        nch = (ncols + _CV - 1) // _CV

        def _startf(k, buf, sem):
            v0c = jnp.minimum(lo + k * _CV, aend - _CV)
            for fs in range(_NSWEEP):
                @pl.when(fi == fs)
                def _():
                    pltpu.async_copy(tabs_t[fs].at[:, pl.ds(v0c, _CV)],
                                     buf, sem)

        def _waitf(k, buf, sem):
            v0c = jnp.minimum(lo + k * _CV, aend - _CV)
            for fs in range(_NSWEEP):
                @pl.when(fi == fs)
                def _():
                    pltpu.make_async_copy(
                        tabs_t[fs].at[:, pl.ds(v0c, _CV)], buf, sem).wait()

        _startf(0, cbuf, gsem0)
        pltpu.sync_copy(idx_scan.at[fi], sbuf)

        def _scan_row(row, cnt):
            for m in range(8):
                iv = sbuf[row, pl.ds(m * 16, 16)]
                msk = (iv >= lo) & (iv < hi)
                pc = plsc.all_reduce_population_count(msk)[0]

                @pl.when(pc > 0)
                def _(cnt=cnt):
                    bvec = row * 128 + (m * 16 + iota)
                    cc = jnp.minimum(cnt, _CAPC)
                    plsc.store_compressed(hit_r.at[pl.ds(cc, 16)], iv,
                                          mask=msk)
                    plsc.store_compressed(hit_b.at[pl.ds(cc, 16)], bvec,
                                          mask=msk)
                cnt = cnt + pc
            return cnt

        cnt = pl.loop(0, 128, init_carry=jnp.int32(0))(_scan_row)
        nhv = (cnt + 15) // 16

        def _process(k, buf):
            v0 = lo + k * _CV
            vhi = jnp.minimum(v0 + _CV, hi)
            v0c = jnp.minimum(v0, aend - _CV)

            def _compress(hv, cc):
                rv = hit_r[pl.ds(hv * 16, 16)]
                bv = hit_b[pl.ds(hv * 16, 16)]
                m2 = ((hv * 16 + iota) < cnt) & (rv >= v0) & (rv < vhi)
                pc2 = plsc.all_reduce_population_count(m2)[0]

                @pl.when(pc2 > 0)
                def _():
                    cx = jnp.minimum(cc, _CAPC)
                    plsc.store_compressed(ch_r.at[pl.ds(cx, 16)], rv,
                                          mask=m2)
                    plsc.store_compressed(ch_b.at[pl.ds(cx, 16)], bv,
                                          mask=m2)
                return cc + pc2

            cc = pl.loop(0, nhv, init_carry=jnp.int32(0))(_compress)
            nvv = (cc + 15) // 16

            @pl.loop(0, nvv)
            def _extract(dv):
                @pl.when((dv > 0) & (dv % 4 == 0))
                def _():
                    pltpu.make_async_copy(drain8, drain8_v, osem).wait()
                    pltpu.make_async_copy(drain8, drain8_v, osem).wait()
                rv2 = ch_r[pl.ds(dv * 16, 16)]
                bv2 = ch_b[pl.ds(dv * 16, 16)]
                for l in range(16):
                    @pl.when((dv * 16 + l) < cc)
                    def _():
                        r = rv2[l]
                        b = bv2[l]
                        csp = jnp.full((16,), 0, jnp.int32) + (r - v0c)
                        x0 = plsc.load_gather(buf, [iota, csp])
                        x1 = plsc.load_gather(buf, [iota + 16, csp])
                        h = (dv * 16 + l) % _HS
                        hstage[h, pl.ds(0, 16)] = x0
                        hstage[h, pl.ds(16, 16)] = x1
                        pltpu.async_copy(
                            hstage.at[h],
                            out.at[b, pl.ds(fi * _EMBED_DIM, _EMBED_DIM)],
                            osem)

            md = jnp.maximum(nvv - 1, 0) // 4
            rem = cc - 64 * md

            @pl.loop(0, rem)
            def _dr(i):
                pltpu.make_async_copy(out.at[0, pl.ds(0, _EMBED_DIM)],
                                      hstage.at[0], osem).wait()

        @pl.loop(0, (nch + 1) // 2)
        def _pair(kk):
            k0 = 2 * kk

            @pl.when(k0 + 1 < nch)
            def _():
                _startf(k0 + 1, cbuf1, gsem1)
            _waitf(k0, cbuf, gsem0)
            _process(k0, cbuf)

            @pl.when(k0 + 2 < nch)
            def _():
                _startf(k0 + 2, cbuf, gsem0)

            @pl.when(k0 + 1 < nch)
            def _():
                _waitf(k0 + 1, cbuf1, gsem1)
                _process(k0 + 1, cbuf1)


@jax.jit
def _encoder(tabs_t, smalls, tail_all, idx_all, idx_scan, drain8, drain32):
    grid_kernel = pl.kernel(
        _body,
        out_type=jax.ShapeDtypeStruct((_BATCH, _OUT_W), jnp.float32),
        mesh=plsc.VectorSubcoreMesh(core_axis_name="c", subcore_axis_name="s"),
        compiler_params=pltpu.CompilerParams(needs_layout_passes=False),
        scratch_types=[
            pltpu.VMEM((_IDX_ROWS, 128), jnp.int32),
            pltpu.VMEM((_SCHUNK, _NSMALL * _EMBED_DIM), jnp.float32),
            pltpu.VMEM((128, 128), jnp.int32),
            pltpu.VMEM((_CAP,), jnp.int32),
            pltpu.VMEM((_CAP,), jnp.int32),
            pltpu.VMEM((_CAP,), jnp.int32),
            pltpu.VMEM((_CAP,), jnp.int32),
            pltpu.VMEM((32, _CV), jnp.float32),
            pltpu.VMEM((32, _CV), jnp.float32),
            pltpu.VMEM((_HS, _EMBED_DIM), jnp.float32),
            pltpu.VMEM((8, 128), jnp.float32),
            pltpu.VMEM((32, 128), jnp.float32),
            pltpu.SemaphoreType.DMA,
            pltpu.SemaphoreType.DMA,
            pltpu.SemaphoreType.DMA,
            pltpu.SemaphoreType.DMA,
        ],
    )
    return grid_kernel(*tabs_t, *smalls, tail_all, idx_all, idx_scan,
                       drain8, drain32)


def kernel(table_0, table_1, table_2, table_3, table_4, table_5, table_6,
           table_7, table_8, table_9, table_10, table_11, table_12, table_13,
           table_14, table_15, table_16, table_17, table_18, table_19,
           table_20, table_21, table_22, table_23, table_24, table_25,
           idx_0, idx_1, idx_2, idx_3, idx_4, idx_5, idx_6, idx_7, idx_8,
           idx_9, idx_10, idx_11, idx_12, idx_13, idx_14, idx_15, idx_16,
           idx_17, idx_18, idx_19, idx_20, idx_21, idx_22, idx_23, idx_24,
           idx_25):
    tables = (table_0, table_1, table_2, table_3, table_4, table_5, table_6,
              table_7, table_8, table_9, table_10, table_11, table_12,
              table_13, table_14, table_15, table_16, table_17, table_18,
              table_19, table_20, table_21, table_22, table_23, table_24,
              table_25)
    idxs = (idx_0, idx_1, idx_2, idx_3, idx_4, idx_5, idx_6, idx_7, idx_8,
            idx_9, idx_10, idx_11, idx_12, idx_13, idx_14, idx_15, idx_16,
            idx_17, idx_18, idx_19, idx_20, idx_21, idx_22, idx_23, idx_24,
            idx_25)
    tabs_t = tuple(jnp.transpose(tables[f]) for f in range(_NSWEEP))
    smalls = tuple(tables[f] for f in range(_NSWEEP, _NUM_FEATS))
    tail_all = jnp.concatenate(
        [tables[f][_AEND[f]:] for f in range(_NSWEEP)], axis=0)
    idx_all = jnp.transpose(
        jnp.stack(idxs).reshape(_NUM_FEATS, _NW, _BPW), (1, 0, 2)
    ).reshape(_NW, _IDX_ROWS, 128)
    idx_scan = jnp.stack(idxs[:_NSWEEP]).reshape(_NSWEEP, 128, 128)
    drain8 = jnp.zeros((8, 128), jnp.float32)
    drain32 = jnp.zeros((32, 128), jnp.float32)
    return _encoder(tabs_t, smalls, tail_all, idx_all, idx_scan,
                    drain8, drain32)
